# Initial kernel scaffold; baseline (speedup 1.0000x reference)
import jax
import jax.numpy as jnp
from jax.experimental import pallas as pl


def kernel(x, edge_attr, params, edge_index, batch):
    raise NotImplementedError("write your pallas kernel here")


# speedup vs baseline: 3.3400x; 3.3400x over previous
import functools

import jax
import jax.numpy as jnp
from jax import lax
from jax.experimental import pallas as pl
from jax.experimental.pallas import tpu as pltpu
from jax.experimental.pallas import tpu_sc as plsc

N = 10000
E = 320000
IN_DIM = 128
HID = 64
HL = 32
G = 64

NC = 2
NS = 16
NW = NC * NS

EW = 10240
E_PAD = NW * EW
SCH = 1024
NSCH = EW // SCH
N_PAD = 10240
RPT = N_PAD // NS

def _s1_gather_body(a_hbm, b_hbm, dst_hbm, src_hbm, ea_hbm, v1_hbm, pre_hbm,
                    idxd, idxs, eav, bufa, bufb, v1v, sema, semb):
    wid = lax.axis_index("s") * NC + lax.axis_index("c")
    base_w = wid * EW
    pltpu.sync_copy(v1_hbm, v1v)
    v1a = v1v[pl.ds(0, 16)]
    v1b = v1v[pl.ds(16, 16)]

    def superchunk(i, _):
        base = pl.multiple_of(base_w + i * SCH, SCH)
        rbase = pl.multiple_of(base // 128, 8)
        pltpu.sync_copy(dst_hbm.at[pl.ds(rbase, SCH // 128)], idxd)
        pltpu.sync_copy(src_hbm.at[pl.ds(rbase, SCH // 128)], idxs)
        pltpu.sync_copy(ea_hbm.at[pl.ds(base, SCH)], eav)
        for j in range(SCH // 128):
            pltpu.async_copy(a_hbm.at[idxd.at[j]],
                             bufa.at[pl.ds(j * 128, 128)], sema)
            pltpu.async_copy(b_hbm.at[idxs.at[j]],
                             bufb.at[pl.ds(j * 128, 128)], semb)
        pltpu.make_async_copy(a_hbm.at[idxd.at[0]], bufa, sema).wait()
        pltpu.make_async_copy(b_hbm.at[idxs.at[0]], bufb, semb).wait()

        def group(g, _):
            j0 = g * 16
            ea16 = eav[pl.ds(j0, 16)]
            for i in range(16):
                j = j0 + i
                s = ea16[i]
                bufa[j, pl.ds(0, 16)] = (bufa[j, pl.ds(0, 16)]
                                         + bufb[j, pl.ds(0, 16)] + s * v1a)
                bufa[j, pl.ds(16, 16)] = (bufa[j, pl.ds(16, 16)]
                                          + bufb[j, pl.ds(16, 16)] + s * v1b)
            return 0

        lax.fori_loop(0, SCH // 16, group, 0)
        pltpu.sync_copy(bufa, pre_hbm.at[pl.ds(base, SCH)])
        return 0

    lax.fori_loop(0, NSCH, superchunk, 0)


def _s2_scatter_body(m_hbm, dst_hbm, out_hbm, idxd, mbuf, zbuf, acc, sem):
    cid = lax.axis_index("c")
    sid = lax.axis_index("s")
    wid = cid * NS + sid

    def zrow(j, _):
        zbuf[j, pl.ds(0, 16)] = jnp.zeros((16,), jnp.float32)
        zbuf[j, pl.ds(16, 16)] = jnp.zeros((16,), jnp.float32)
        zbuf[j, pl.ds(32, 16)] = jnp.zeros((16,), jnp.float32)
        zbuf[j, pl.ds(48, 16)] = jnp.zeros((16,), jnp.float32)
        return 0

    lax.fori_loop(0, 128, zrow, 0)

    def zcopy(k, _):
        pltpu.sync_copy(zbuf, acc.at[pl.ds(pl.multiple_of(sid * RPT + k * 128, 128), 128)])
        return 0

    lax.fori_loop(0, RPT // 128, zcopy, 0)
    plsc.subcore_barrier()

    def superchunk(i, _):
        base = pl.multiple_of(wid * EW + i * SCH, SCH)
        pltpu.sync_copy(dst_hbm.at[pl.ds(pl.multiple_of(base // 128, 8), SCH // 128)], idxd)
        pltpu.sync_copy(m_hbm.at[pl.ds(base, SCH)], mbuf)
        for j in range(SCH // 128):
            pltpu.sync_copy(mbuf.at[pl.ds(j * 128, 128)],
                            acc.at[idxd.at[j]], add=True)
        return 0

    lax.fori_loop(0, NSCH, superchunk, 0)
    plsc.subcore_barrier()
    srow = pl.multiple_of(sid * RPT, RPT)
    pltpu.sync_copy(acc.at[pl.ds(srow, RPT)],
                    out_hbm.at[cid, pl.ds(srow, RPT)])


@functools.cache
def _sc_kernels():
    mesh = plsc.VectorSubcoreMesh(
        core_axis_name="c", subcore_axis_name="s",
        num_cores=NC, num_subcores=NS)
    scp = pltpu.CompilerParams(use_tc_tiling_on_sc=False)
    s1 = pl.kernel(
        _s1_gather_body,
        out_type=jax.ShapeDtypeStruct((E_PAD, HL), jnp.float32),
        mesh=mesh,
        compiler_params=scp,
        scratch_types=[
            pltpu.VMEM((SCH // 128, 128), jnp.int32),
            pltpu.VMEM((SCH // 128, 128), jnp.int32),
            pltpu.VMEM((SCH,), jnp.float32),
            pltpu.VMEM((SCH, HL), jnp.float32),
            pltpu.VMEM((SCH, HL), jnp.float32),
            pltpu.VMEM((HL,), jnp.float32),
            pltpu.SemaphoreType.DMA,
            pltpu.SemaphoreType.DMA,
        ])
    s2 = pl.kernel(
        _s2_scatter_body,
        out_type=jax.ShapeDtypeStruct((NC, N_PAD, HID), jnp.float32),
        mesh=mesh,
        compiler_params=scp,
        scratch_types=[
            pltpu.VMEM((SCH // 128, 128), jnp.int32),
            pltpu.VMEM((SCH, HID), jnp.float32),
            pltpu.VMEM((128, HID), jnp.float32),
            pltpu.VMEM_SHARED((N_PAD, HID), jnp.float32),
            pltpu.SemaphoreType.DMA,
        ])
    return s1, s2


def _k1_body(x_ref, winT, bin_, wdT, wsT, v0, h_ref, a_ref, b_ref):
    h = jnp.dot(x_ref[...], winT[...],
                preferred_element_type=jnp.float32) + bin_[...]
    h_ref[...] = h
    a_ref[...] = jnp.dot(h, wdT[...], preferred_element_type=jnp.float32) + v0[...]
    b_ref[...] = jnp.dot(h, wsT[...], preferred_element_type=jnp.float32)


def _k2_body(pre_ref, w2T, b2, s2, be2, m_ref, *, eb):
    pid = pl.program_id(0)
    z = jax.nn.relu(pre_ref[...])
    u = jnp.dot(z, w2T[...], preferred_element_type=jnp.float32) + b2[...]
    m = s2[...] * jax.nn.relu(u) + be2[...]
    rows = pid * eb + lax.broadcasted_iota(jnp.int32, (eb, 1), 0)
    m_ref[...] = jnp.where(rows < E, m, 0.0)


def _k3_body(h_ref, ag_ref, uhT, uaT, v0u, u2T, b2u, s2u, be2u,
             wdT, wsT, v0n, hn_ref, a_ref, b_ref):
    h = h_ref[...]
    aggr = ag_ref[0, :N, :] + ag_ref[1, :N, :]
    t = jax.nn.relu(jnp.dot(h, uhT[...], preferred_element_type=jnp.float32)
                    + jnp.dot(aggr, uaT[...], preferred_element_type=jnp.float32)
                    + v0u[...])
    u2 = jnp.dot(t, u2T[...], preferred_element_type=jnp.float32) + b2u[...]
    hn = h + s2u[...] * jax.nn.relu(u2) + be2u[...]
    hn_ref[...] = hn
    a_ref[...] = jnp.dot(hn, wdT[...], preferred_element_type=jnp.float32) + v0n[...]
    b_ref[...] = jnp.dot(hn, wsT[...], preferred_element_type=jnp.float32)


def _k4_body(h_ref, batch_ref, woutT, bout, out_ref):
    b = batch_ref[...]
    mask = (b == lax.broadcasted_iota(jnp.int32, (G, N), 0)).astype(jnp.float32)
    sums = jnp.dot(mask, h_ref[...], preferred_element_type=jnp.float32)
    counts = jnp.sum(mask, axis=1, keepdims=True)
    pooled = sums / jnp.maximum(counts, 1.0)
    out = jnp.dot(pooled, woutT[...], preferred_element_type=jnp.float32) + bout[...]
    out_ref[...] = jax.nn.relu(out)


def _fold_mlp(p):
    c = 1.0 / jnp.sqrt(jnp.float32(1.0 + 1e-5))
    s1 = p["g1"] * c
    w1 = p["lin1"]["w"]
    v0 = s1 * p["lin1"]["b"] + p["be1"]
    s2 = p["g2"] * c
    return s1, w1, v0, p["lin2"]["w"].T, p["lin2"]["b"], s2, p["be2"]


def kernel(x, edge_attr, params, edge_index, batch):
    pad = E_PAD - E
    dstp = jnp.concatenate(
        [edge_index[1], jnp.zeros((pad,), jnp.int32)]).reshape(E_PAD // 128, 128)
    srcp = jnp.concatenate(
        [edge_index[0], jnp.zeros((pad,), jnp.int32)]).reshape(E_PAD // 128, 128)
    eap = jnp.concatenate([edge_attr.reshape(-1),
                           jnp.zeros((pad,), jnp.float32)])

    layers = []
    for cp in params["convs"]:
        s1, w1, v0, w2T, b2, s2, be2 = _fold_mlp(cp["msg"])
        wdT = (w1[:, :HID] * s1[:, None]).T
        wsT = (w1[:, HID:2 * HID] * s1[:, None]).T
        v1 = s1 * w1[:, 2 * HID]
        s1u, u1, v0u, u2T, b2u, s2u, be2u = _fold_mlp(cp["upd"])
        uhT = (u1[:, :HID] * s1u[:, None]).T
        uaT = (u1[:, HID:] * s1u[:, None]).T
        layers.append(dict(
            wdT=wdT, wsT=wsT, v1=v1, v0=v0.reshape(1, HL),
            w2T=w2T, b2=b2.reshape(1, HID), s2=s2.reshape(1, HID),
            be2=be2.reshape(1, HID),
            uhT=uhT, uaT=uaT, v0u=v0u.reshape(1, HL), u2T=u2T,
            b2u=b2u.reshape(1, HID), s2u=s2u.reshape(1, HID),
            be2u=be2u.reshape(1, HID)))

    winT = params["lin_in"]["w"].T
    bin_ = params["lin_in"]["b"].reshape(1, HID)
    woutT = params["lin_out"]["w"].T
    bout = params["lin_out"]["b"].reshape(1, 1)

    f32 = jnp.float32
    h, a, b = pl.pallas_call(
        _k1_body,
        out_shape=[jax.ShapeDtypeStruct((N, HID), f32),
                   jax.ShapeDtypeStruct((N, HL), f32),
                   jax.ShapeDtypeStruct((N, HL), f32)],
    )(x, winT, bin_, layers[0]["wdT"], layers[0]["wsT"], layers[0]["v0"])

    s1_gather, s2_scatter = _sc_kernels()
    EB = 16384
    for li, ly in enumerate(layers):
        pre = s1_gather(a, b, dstp, srcp, eap, ly["v1"])
        m = pl.pallas_call(
            functools.partial(_k2_body, eb=EB),
            grid=(E_PAD // EB,),
            in_specs=[
                pl.BlockSpec((EB, HL), lambda i: (i, 0)),
                pl.BlockSpec((HL, HID), lambda i: (0, 0)),
                pl.BlockSpec((1, HID), lambda i: (0, 0)),
                pl.BlockSpec((1, HID), lambda i: (0, 0)),
                pl.BlockSpec((1, HID), lambda i: (0, 0)),
            ],
            out_specs=pl.BlockSpec((EB, HID), lambda i: (i, 0)),
            out_shape=jax.ShapeDtypeStruct((E_PAD, HID), f32),
        )(pre, ly["w2T"], ly["b2"], ly["s2"], ly["be2"])
        ag = s2_scatter(m, dstp)
        nxt = layers[li + 1] if li + 1 < len(layers) else layers[0]
        h, a, b = pl.pallas_call(
            _k3_body,
            out_shape=[jax.ShapeDtypeStruct((N, HID), f32),
                       jax.ShapeDtypeStruct((N, HL), f32),
                       jax.ShapeDtypeStruct((N, HL), f32)],
        )(h, ag, ly["uhT"], ly["uaT"], ly["v0u"], ly["u2T"], ly["b2u"],
          ly["s2u"], ly["be2u"], nxt["wdT"], nxt["wsT"], nxt["v0"])

    out = pl.pallas_call(
        _k4_body,
        out_shape=jax.ShapeDtypeStruct((G, 1), f32),
    )(h, batch.reshape(1, N), woutT, bout)
    return out.reshape(-1)


# speedup vs baseline: 6.5715x; 1.9675x over previous
import functools

import jax
import jax.numpy as jnp
import numpy as np
from jax import lax
from jax.experimental import pallas as pl
from jax.experimental.pallas import tpu as pltpu
from jax.experimental.pallas import tpu_sc as plsc

N = 10000
E = 320000
IN_DIM = 128
HID = 64
HL = 32
G = 64

NC = 2
NS = 16
NW = NC * NS

EW = 10240
E_PAD = NW * EW
SCH = 512
NSCH = EW // SCH
HR = 2 * E_PAD
HRW = HR // NW
SCH2 = 1024
NSCH2 = HRW // SCH2
N_PAD = 10240
RPT = N_PAD // NS
EB = 16384


def _s1_gather_body(a_hbm, b_hbm, dst_hbm, src_hbm, pre_hbm,
                    idxd0, idxs0, idxd1, idxs1,
                    bufa0, bufb0, bufa1, bufb1, bufo0, bufo1,
                    sga0, sgb0, sga1, sgb1, so0, so1):
    wid = lax.axis_index("s") * NC + lax.axis_index("c")
    base_w = wid * EW
    idxd = [idxd0, idxd1]
    idxs = [idxs0, idxs1]
    bufa = [bufa0, bufa1]
    bufb = [bufb0, bufb1]
    bufo = [bufo0, bufo1]
    sga = [sga0, sga1]
    sgb = [sgb0, sgb1]
    so = [so0, so1]

    def fetch(i, s):
        base = pl.multiple_of(base_w + i * SCH, SCH)
        rbase = pl.multiple_of(base // 128, 4)
        pltpu.sync_copy(dst_hbm.at[pl.ds(rbase, SCH // 128)], idxd[s])
        pltpu.sync_copy(src_hbm.at[pl.ds(rbase, SCH // 128)], idxs[s])
        for j in range(SCH // 128):
            pltpu.async_copy(a_hbm.at[idxd[s].at[j]],
                             bufa[s].at[pl.ds(j * 128, 128)], sga[s])
            pltpu.async_copy(b_hbm.at[idxs[s].at[j]],
                             bufb[s].at[pl.ds(j * 128, 128)], sgb[s])

    def wait_fetch(s):
        pltpu.make_async_copy(a_hbm.at[pl.ds(0, SCH)], bufa[s], sga[s]).wait()
        pltpu.make_async_copy(b_hbm.at[pl.ds(0, SCH)], bufb[s], sgb[s]).wait()

    def combine(s):
        ba, bb, bo = bufa[s], bufb[s], bufo[s]

        def group(g, _):
            j0 = g * 16
            r0 = g * 4
            for i in range(16):
                j = j0 + i
                c = (i % 4) * 32
                r = r0 + i // 4
                bo[r, pl.ds(c, 16)] = (ba[j, pl.ds(0, 16)]
                                       + bb[j, pl.ds(0, 16)])
                bo[r, pl.ds(c + 16, 16)] = (ba[j, pl.ds(16, 16)]
                                            + bb[j, pl.ds(16, 16)])
            return 0

        lax.fori_loop(0, SCH // 16, group, 0)

    def put(i, s):
        base = pl.multiple_of(base_w + i * SCH, SCH)
        pltpu.async_copy(bufo[s], pre_hbm.at[pl.ds(base // 4, SCH // 4)],
                         so[s])

    def drain_put(s):
        pltpu.make_async_copy(pre_hbm.at[pl.ds(0, SCH // 4)],
                              bufo[s], so[s]).wait()

    fetch(0, 0)

    def pair(ii, _):
        i = 2 * ii
        fetch(i + 1, 1)
        wait_fetch(0)
        combine(0)

        @pl.when(ii > 0)
        def _():
            drain_put(0)

        put(i, 0)

        @pl.when(ii + 1 < NSCH // 2)
        def _():
            fetch(i + 2, 0)

        wait_fetch(1)
        combine(1)

        @pl.when(ii > 0)
        def _():
            drain_put(1)

        put(i + 1, 1)
        return 0

    lax.fori_loop(0, NSCH // 2, pair, 0)
    drain_put(0)
    drain_put(1)


def _s2_scatter_body(m_hbm, dst_hbm, out_hbm,
                     idx0, idx1, mbuf0, mbuf1, zbuf, acc,
                     sm0, sm1):
    cid = lax.axis_index("c")
    sid = lax.axis_index("s")
    wid = cid * NS + sid
    base_w = wid * HRW
    idx = [idx0, idx1]
    mbuf = [mbuf0, mbuf1]
    sm = [sm0, sm1]

    def zrow(j, _):
        zbuf[j, pl.ds(0, 16)] = jnp.zeros((16,), jnp.float32)
        zbuf[j, pl.ds(16, 16)] = jnp.zeros((16,), jnp.float32)
        return 0

    lax.fori_loop(0, 256, zrow, 0)

    def zcopy(k, _):
        pltpu.sync_copy(
            zbuf,
            acc.at[pl.ds(pl.multiple_of(sid * 2 * RPT + k * 256, 256), 256)])
        return 0

    lax.fori_loop(0, 2 * RPT // 256, zcopy, 0)
    plsc.subcore_barrier()

    def fetch(i, s):
        base = pl.multiple_of(base_w + i * SCH2, SCH2)
        pltpu.sync_copy(dst_hbm.at[pl.ds(pl.multiple_of(base // 128, 8),
                                         SCH2 // 128)], idx[s])
        pltpu.async_copy(m_hbm.at[pl.ds(base, SCH2)], mbuf[s], sm[s])

    def wait_fetch(s):
        pltpu.make_async_copy(m_hbm.at[pl.ds(0, SCH2)], mbuf[s],
                              sm[s]).wait()

    def scatter(s):
        for j in range(SCH2 // 128):
            pltpu.sync_copy(mbuf[s].at[pl.ds(j * 128, 128)],
                            acc.at[idx[s].at[j]], add=True)

    fetch(0, 0)

    def pair(ii, _):
        i = 2 * ii
        fetch(i + 1, 1)
        wait_fetch(0)
        scatter(0)

        @pl.when(ii + 1 < NSCH2 // 2)
        def _():
            fetch(i + 2, 0)

        wait_fetch(1)
        scatter(1)
        return 0

    lax.fori_loop(0, NSCH2 // 2, pair, 0)
    plsc.subcore_barrier()
    srow = pl.multiple_of(sid * 2 * RPT, 2 * RPT)
    pltpu.sync_copy(acc.at[pl.ds(srow, 2 * RPT)],
                    out_hbm.at[cid, pl.ds(srow, 2 * RPT)])


@functools.cache
def _sc_kernels():
    mesh = plsc.VectorSubcoreMesh(
        core_axis_name="c", subcore_axis_name="s",
        num_cores=NC, num_subcores=NS)
    scp = pltpu.CompilerParams(use_tc_tiling_on_sc=False)
    i32, f32 = jnp.int32, jnp.float32
    s1 = pl.kernel(
        _s1_gather_body,
        out_type=jax.ShapeDtypeStruct((E_PAD // 4, 128), f32),
        mesh=mesh,
        compiler_params=scp,
        scratch_types=(
            [pltpu.VMEM((SCH // 128, 128), i32) for _ in range(4)]
            + [pltpu.VMEM((SCH, HL), f32) for _ in range(4)]
            + [pltpu.VMEM((SCH // 4, 128), f32) for _ in range(2)]
            + [pltpu.SemaphoreType.DMA for _ in range(6)]
        ))
    s2 = pl.kernel(
        _s2_scatter_body,
        out_type=jax.ShapeDtypeStruct((NC, 2 * N_PAD, HL), f32),
        mesh=mesh,
        compiler_params=scp,
        scratch_types=(
            [pltpu.VMEM((SCH2 // 128, 128), i32) for _ in range(2)]
            + [pltpu.VMEM((SCH2, HL), f32) for _ in range(2)]
            + [pltpu.VMEM((256, HL), f32)]
            + [pltpu.VMEM_SHARED((2 * N_PAD, HL), f32)]
            + [pltpu.SemaphoreType.DMA for _ in range(2)]
        ))
    return s1, s2


def _k1_body(x_ref, winT, bin_, wdT, wsT, v0, h_ref, a_ref, b_ref):
    h = jnp.dot(x_ref[...], winT[...],
                preferred_element_type=jnp.float32) + bin_[...]
    h_ref[...] = h
    a_ref[...] = jnp.dot(h, wdT[...], preferred_element_type=jnp.float32) + v0[...]
    b_ref[...] = jnp.dot(h, wsT[...], preferred_element_type=jnp.float32)


def _k2_body(pre_ref, ea_ref, kmat, we, wo, b2_, s2_, be2_, m_ref):
    pid = pl.program_id(0)
    z = jax.nn.relu(pre_ref[...]
                    + jnp.dot(ea_ref[...], kmat[...],
                              preferred_element_type=jnp.float32))
    rows = pid * EB + 4 * lax.broadcasted_iota(jnp.int32, (EB // 4, 1), 0)
    valid = rows < E
    meven = s2_[...] * jax.nn.relu(
        jnp.dot(z, we[...], preferred_element_type=jnp.float32)
        + b2_[...]) + be2_[...]
    modd = s2_[...] * jax.nn.relu(
        jnp.dot(z, wo[...], preferred_element_type=jnp.float32)
        + b2_[...]) + be2_[...]
    m_ref[:EB // 4, :] = jnp.where(valid, meven, 0.0)
    m_ref[EB // 4:, :] = jnp.where(valid, modd, 0.0)


def _k3_body(h_ref, ag_ref, uhT, uaT, v0u, u2T, b2u, s2u, be2u,
             wdT, wsT, v0n, hn_ref, a_ref, b_ref):
    h = h_ref[...]
    aggr = ag_ref[0, :N, :] + ag_ref[1, :N, :]
    t = jax.nn.relu(jnp.dot(h, uhT[...], preferred_element_type=jnp.float32)
                    + jnp.dot(aggr, uaT[...], preferred_element_type=jnp.float32)
                    + v0u[...])
    u2 = jnp.dot(t, u2T[...], preferred_element_type=jnp.float32) + b2u[...]
    hn = h + s2u[...] * jax.nn.relu(u2) + be2u[...]
    hn_ref[...] = hn
    a_ref[...] = jnp.dot(hn, wdT[...], preferred_element_type=jnp.float32) + v0n[...]
    b_ref[...] = jnp.dot(hn, wsT[...], preferred_element_type=jnp.float32)


def _k4_body(h_ref, batch_ref, woutT, bout, out_ref):
    b = batch_ref[...]
    mask = (b == lax.broadcasted_iota(jnp.int32, (G, N), 0)).astype(jnp.float32)
    sums = jnp.dot(mask, h_ref[...], preferred_element_type=jnp.float32)
    counts = jnp.sum(mask, axis=1, keepdims=True)
    pooled = sums / jnp.maximum(counts, 1.0)
    out = jnp.dot(pooled, woutT[...], preferred_element_type=jnp.float32) + bout[...]
    out_ref[...] = jax.nn.relu(out)


def _fold_mlp(p):
    c = 1.0 / jnp.sqrt(jnp.float32(1.0 + 1e-5))
    s1 = p["g1"] * c
    w1 = p["lin1"]["w"]
    v0 = s1 * p["lin1"]["b"] + p["be1"]
    s2 = p["g2"] * c
    return s1, w1, v0, p["lin2"]["w"].T, p["lin2"]["b"], s2, p["be2"]


def _edge_perm():
    r = np.arange(EB // 4)
    evens = np.stack([4 * r, 4 * r + 1], 1).reshape(-1)
    odds = np.stack([4 * r + 2, 4 * r + 3], 1).reshape(-1)
    block_order = np.concatenate([evens, odds])
    return (np.arange(E_PAD // EB)[:, None] * EB
            + block_order[None, :]).reshape(-1)


_EDGE_PERM = _edge_perm()


def kernel(x, edge_attr, params, edge_index, batch):
    pad = E_PAD - E
    dst1 = jnp.concatenate([edge_index[1], jnp.zeros((pad,), jnp.int32)])
    src1 = jnp.concatenate([edge_index[0], jnp.zeros((pad,), jnp.int32)])
    dstp = dst1.reshape(E_PAD // 128, 128)
    srcp = src1.reshape(E_PAD // 128, 128)
    eap = jnp.concatenate([edge_attr.reshape(-1),
                           jnp.zeros((pad,), jnp.float32)])
    ea4 = eap.reshape(E_PAD // 4, 4)
    dperm = dst1[jnp.asarray(_EDGE_PERM)]
    dst2 = (2 * dperm[:, None]
            + jnp.arange(2, dtype=jnp.int32)[None, :]).reshape(HR // 128, 128)

    layers = []
    for cp in params["convs"]:
        s1, w1, v0, w2T, b2, s2, be2 = _fold_mlp(cp["msg"])
        wdT = (w1[:, :HID] * s1[:, None]).T
        wsT = (w1[:, HID:2 * HID] * s1[:, None]).T
        v1 = s1 * w1[:, 2 * HID]
        kmat = jnp.zeros((4, 128), jnp.float32)
        for i in range(4):
            kmat = kmat.at[i, 32 * i:32 * (i + 1)].set(v1)
        we = jnp.zeros((128, 128), jnp.float32)
        we = we.at[0:32, 0:64].set(w2T).at[32:64, 64:128].set(w2T)
        wo = jnp.zeros((128, 128), jnp.float32)
        wo = wo.at[64:96, 0:64].set(w2T).at[96:128, 64:128].set(w2T)
        s1u, u1, v0u, u2T, b2u, s2u, be2u = _fold_mlp(cp["upd"])
        uhT = (u1[:, :HID] * s1u[:, None]).T
        uaT = (u1[:, HID:] * s1u[:, None]).T
        two = lambda v: jnp.concatenate([v, v]).reshape(1, 128)
        layers.append(dict(
            wdT=wdT, wsT=wsT, v0=v0.reshape(1, HL), kmat=kmat, we=we, wo=wo,
            b2=two(b2), s2=two(s2), be2=two(be2),
            uhT=uhT, uaT=uaT, v0u=v0u.reshape(1, HL), u2T=u2T,
            b2u=b2u.reshape(1, HID), s2u=s2u.reshape(1, HID),
            be2u=be2u.reshape(1, HID)))

    winT = params["lin_in"]["w"].T
    bin_ = params["lin_in"]["b"].reshape(1, HID)
    woutT = params["lin_out"]["w"].T
    bout = params["lin_out"]["b"].reshape(1, 1)

    f32 = jnp.float32
    h, a, b = pl.pallas_call(
        _k1_body,
        out_shape=[jax.ShapeDtypeStruct((N, HID), f32),
                   jax.ShapeDtypeStruct((N, HL), f32),
                   jax.ShapeDtypeStruct((N, HL), f32)],
    )(x, winT, bin_, layers[0]["wdT"], layers[0]["wsT"], layers[0]["v0"])

    s1_gather, s2_scatter = _sc_kernels()
    for li, ly in enumerate(layers):
        pre4 = s1_gather(a, b, dstp, srcp)
        m2 = pl.pallas_call(
            _k2_body,
            grid=(E_PAD // EB,),
            in_specs=[
                pl.BlockSpec((EB // 4, 128), lambda i: (i, 0)),
                pl.BlockSpec((EB // 4, 4), lambda i: (i, 0)),
                pl.BlockSpec((4, 128), lambda i: (0, 0)),
                pl.BlockSpec((128, 128), lambda i: (0, 0)),
                pl.BlockSpec((128, 128), lambda i: (0, 0)),
                pl.BlockSpec((1, 128), lambda i: (0, 0)),
                pl.BlockSpec((1, 128), lambda i: (0, 0)),
                pl.BlockSpec((1, 128), lambda i: (0, 0)),
            ],
            out_specs=pl.BlockSpec((EB // 2, 128), lambda i: (i, 0)),
            out_shape=jax.ShapeDtypeStruct((E_PAD // 2, 128), f32),
        )(pre4, ea4, ly["kmat"], ly["we"], ly["wo"], ly["b2"], ly["s2"],
          ly["be2"])
        ag = s2_scatter(m2.reshape(HR, HL), dst2)
        nxt = layers[li + 1] if li + 1 < len(layers) else layers[0]
        h, a, b = pl.pallas_call(
            _k3_body,
            out_shape=[jax.ShapeDtypeStruct((N, HID), f32),
                       jax.ShapeDtypeStruct((N, HL), f32),
                       jax.ShapeDtypeStruct((N, HL), f32)],
        )(h, ag.reshape(NC, N_PAD, HID), ly["uhT"], ly["uaT"], ly["v0u"],
          ly["u2T"], ly["b2u"], ly["s2u"], ly["be2u"],
          nxt["wdT"], nxt["wsT"], nxt["v0"])

    out = pl.pallas_call(
        _k4_body,
        out_shape=jax.ShapeDtypeStruct((G, 1), f32),
    )(h, batch.reshape(1, N), woutT, bout)
    return out.reshape(-1)


# speedup vs baseline: 6.9549x; 1.0583x over previous
import functools

import jax
import jax.numpy as jnp
import numpy as np
from jax import lax
from jax.experimental import pallas as pl
from jax.experimental.pallas import tpu as pltpu
from jax.experimental.pallas import tpu_sc as plsc

N = 10000
E = 320000
IN_DIM = 128
HID = 64
HL = 32
G = 64

NC = 2
NS = 16
NW = NC * NS

EW = 10240
E_PAD = NW * EW
SCH = 512
NSCH = EW // SCH
HR = 2 * E_PAD
HRW = HR // NW
SCH2 = 1024
NSCH2 = HRW // SCH2
N_PAD = 10240
RPT = N_PAD // NS
EB = 16384


def _s1_gather_body(a_hbm, b_hbm, dst_hbm, src_hbm, pre_hbm,
                    idxd, idxs,
                    bufa0, bufb0, bufa1, bufb1, bufo0, bufo1,
                    sgi, sga0, sgb0, sga1, sgb1, so0, so1):
    wid = lax.axis_index("s") * NC + lax.axis_index("c")
    base_w = wid * EW
    bufa = [bufa0, bufa1]
    bufb = [bufb0, bufb1]
    bufo = [bufo0, bufo1]
    sga = [sga0, sga1]
    sgb = [sgb0, sgb1]
    so = [so0, so1]

    rb_w = pl.multiple_of(base_w // 128, 8)
    cp1 = pltpu.async_copy(dst_hbm.at[pl.ds(rb_w, EW // 128)], idxd, sgi)
    cp2 = pltpu.async_copy(src_hbm.at[pl.ds(rb_w, EW // 128)], idxs, sgi)
    cp1.wait()
    cp2.wait()

    def fetch(i, s):
        rofs = i * (SCH // 128)
        for j in range(SCH // 128):
            pltpu.async_copy(a_hbm.at[idxd.at[rofs + j]],
                             bufa[s].at[pl.ds(j * 128, 128)], sga[s])
            pltpu.async_copy(b_hbm.at[idxs.at[rofs + j]],
                             bufb[s].at[pl.ds(j * 128, 128)], sgb[s])

    def wait_fetch(s):
        pltpu.make_async_copy(a_hbm.at[pl.ds(0, SCH)], bufa[s], sga[s]).wait()
        pltpu.make_async_copy(b_hbm.at[pl.ds(0, SCH)], bufb[s], sgb[s]).wait()

    def combine(s):
        ba, bb, bo = bufa[s], bufb[s], bufo[s]

        def group(g, _):
            j0 = g * 16
            r0 = g * 4
            for i in range(16):
                j = j0 + i
                c = (i % 4) * 32
                r = r0 + i // 4
                bo[r, pl.ds(c, 16)] = (ba[j, pl.ds(0, 16)]
                                       + bb[j, pl.ds(0, 16)])
                bo[r, pl.ds(c + 16, 16)] = (ba[j, pl.ds(16, 16)]
                                            + bb[j, pl.ds(16, 16)])
            return 0

        lax.fori_loop(0, SCH // 16, group, 0)

    def put(i, s):
        base = pl.multiple_of(base_w + i * SCH, SCH)
        pltpu.async_copy(bufo[s], pre_hbm.at[pl.ds(base // 4, SCH // 4)],
                         so[s])

    def drain_put(s):
        pltpu.make_async_copy(pre_hbm.at[pl.ds(0, SCH // 4)],
                              bufo[s], so[s]).wait()

    fetch(0, 0)

    def pair(ii, _):
        i = 2 * ii
        fetch(i + 1, 1)
        wait_fetch(0)
        combine(0)

        @pl.when(ii > 0)
        def _():
            drain_put(0)

        put(i, 0)

        @pl.when(ii + 1 < NSCH // 2)
        def _():
            fetch(i + 2, 0)

        wait_fetch(1)
        combine(1)

        @pl.when(ii > 0)
        def _():
            drain_put(1)

        put(i + 1, 1)
        return 0

    lax.fori_loop(0, NSCH // 2, pair, 0)
    drain_put(0)
    drain_put(1)


def _s2_scatter_body(m_hbm, dst_hbm, out_hbm,
                     idx, mbuf0, mbuf1, zbuf, acc,
                     sgi, sm0, sm1, ssc):
    cid = lax.axis_index("c")
    sid = lax.axis_index("s")
    wid = cid * NS + sid
    base_w = wid * HRW
    mbuf = [mbuf0, mbuf1]
    sm = [sm0, sm1]

    rb_w = pl.multiple_of(base_w // 128, 8)
    pltpu.async_copy(dst_hbm.at[pl.ds(rb_w, HRW // 128)], idx, sgi).wait()

    def zrow(j, _):
        zbuf[j, pl.ds(0, 16)] = jnp.zeros((16,), jnp.float32)
        zbuf[j, pl.ds(16, 16)] = jnp.zeros((16,), jnp.float32)
        return 0

    lax.fori_loop(0, 64, zrow, 0)

    def zcopy(k, _):
        pltpu.sync_copy(
            zbuf,
            acc.at[pl.ds(pl.multiple_of(sid * 2 * RPT + k * 64, 64), 64)])
        return 0

    lax.fori_loop(0, 2 * RPT // 64, zcopy, 0)
    plsc.subcore_barrier()

    def fetch(i, s):
        base = pl.multiple_of(base_w + i * SCH2, SCH2)
        pltpu.async_copy(m_hbm.at[pl.ds(base, SCH2)], mbuf[s], sm[s])

    def wait_fetch(s):
        pltpu.make_async_copy(m_hbm.at[pl.ds(0, SCH2)], mbuf[s],
                              sm[s]).wait()

    def scatter(i, s):
        rofs = i * (SCH2 // 128)
        for j in range(SCH2 // 128):
            pltpu.async_copy(mbuf[s].at[pl.ds(j * 128, 128)],
                             acc.at[idx.at[rofs + j]], ssc, add=True)
        pltpu.make_async_copy(m_hbm.at[pl.ds(0, SCH2)], mbuf[s],
                              ssc).wait()

    fetch(0, 0)

    def pair(ii, _):
        i = 2 * ii
        fetch(i + 1, 1)
        wait_fetch(0)
        scatter(i, 0)

        @pl.when(ii + 1 < NSCH2 // 2)
        def _():
            fetch(i + 2, 0)

        wait_fetch(1)
        scatter(i + 1, 1)
        return 0

    lax.fori_loop(0, NSCH2 // 2, pair, 0)
    plsc.subcore_barrier()
    srow = pl.multiple_of(sid * 2 * RPT, 2 * RPT)
    pltpu.sync_copy(acc.at[pl.ds(srow, 2 * RPT)],
                    out_hbm.at[cid, pl.ds(srow, 2 * RPT)])


@functools.cache
def _sc_kernels():
    mesh = plsc.VectorSubcoreMesh(
        core_axis_name="c", subcore_axis_name="s",
        num_cores=NC, num_subcores=NS)
    scp = pltpu.CompilerParams(use_tc_tiling_on_sc=False)
    i32, f32 = jnp.int32, jnp.float32
    s1 = pl.kernel(
        _s1_gather_body,
        out_type=jax.ShapeDtypeStruct((E_PAD // 4, 128), f32),
        mesh=mesh,
        compiler_params=scp,
        scratch_types=(
            [pltpu.VMEM((EW // 128, 128), i32) for _ in range(2)]
            + [pltpu.VMEM((SCH, HL), f32) for _ in range(4)]
            + [pltpu.VMEM((SCH // 4, 128), f32) for _ in range(2)]
            + [pltpu.SemaphoreType.DMA for _ in range(7)]
        ))
    s2 = pl.kernel(
        _s2_scatter_body,
        out_type=jax.ShapeDtypeStruct((NC, 2 * N_PAD, HL), f32),
        mesh=mesh,
        compiler_params=scp,
        scratch_types=(
            [pltpu.VMEM((HRW // 128, 128), i32)]
            + [pltpu.VMEM((SCH2, HL), f32) for _ in range(2)]
            + [pltpu.VMEM((64, HL), f32)]
            + [pltpu.VMEM_SHARED((2 * N_PAD, HL), f32)]
            + [pltpu.SemaphoreType.DMA for _ in range(4)]
        ))
    return s1, s2


def _k1_body(x_ref, winT, bin_, wdT, wsT, v0, h_ref, a_ref, b_ref):
    h = jnp.dot(x_ref[...], winT[...],
                preferred_element_type=jnp.float32) + bin_[...]
    h_ref[...] = h
    a_ref[...] = jnp.dot(h, wdT[...], preferred_element_type=jnp.float32) + v0[...]
    b_ref[...] = jnp.dot(h, wsT[...], preferred_element_type=jnp.float32)


def _k2_body(pre_ref, ea_ref, kmat, we, wo, b2_, s2_, be2_, m_ref):
    pid = pl.program_id(0)
    z = jax.nn.relu(pre_ref[...]
                    + jnp.dot(ea_ref[...], kmat[...],
                              preferred_element_type=jnp.float32))
    rows = pid * EB + 4 * lax.broadcasted_iota(jnp.int32, (EB // 4, 1), 0)
    valid = rows < E
    meven = s2_[...] * jax.nn.relu(
        jnp.dot(z, we[...], preferred_element_type=jnp.float32)
        + b2_[...]) + be2_[...]
    modd = s2_[...] * jax.nn.relu(
        jnp.dot(z, wo[...], preferred_element_type=jnp.float32)
        + b2_[...]) + be2_[...]
    m_ref[:EB // 4, :] = jnp.where(valid, meven, 0.0)
    m_ref[EB // 4:, :] = jnp.where(valid, modd, 0.0)


def _k3_body(h_ref, ag_ref, uhT, uaT, v0u, u2T, b2u, s2u, be2u,
             wdT, wsT, v0n, hn_ref, a_ref, b_ref):
    h = h_ref[...]
    aggr = ag_ref[0, :N, :] + ag_ref[1, :N, :]
    t = jax.nn.relu(jnp.dot(h, uhT[...], preferred_element_type=jnp.float32)
                    + jnp.dot(aggr, uaT[...], preferred_element_type=jnp.float32)
                    + v0u[...])
    u2 = jnp.dot(t, u2T[...], preferred_element_type=jnp.float32) + b2u[...]
    hn = h + s2u[...] * jax.nn.relu(u2) + be2u[...]
    hn_ref[...] = hn
    a_ref[...] = jnp.dot(hn, wdT[...], preferred_element_type=jnp.float32) + v0n[...]
    b_ref[...] = jnp.dot(hn, wsT[...], preferred_element_type=jnp.float32)


def _k4_body(h_ref, batch_ref, woutT, bout, out_ref):
    b = batch_ref[...]
    mask = (b == lax.broadcasted_iota(jnp.int32, (G, N), 0)).astype(jnp.float32)
    sums = jnp.dot(mask, h_ref[...], preferred_element_type=jnp.float32)
    counts = jnp.sum(mask, axis=1, keepdims=True)
    pooled = sums / jnp.maximum(counts, 1.0)
    out = jnp.dot(pooled, woutT[...], preferred_element_type=jnp.float32) + bout[...]
    out_ref[...] = jax.nn.relu(out)


def _fold_mlp(p):
    c = 1.0 / jnp.sqrt(jnp.float32(1.0 + 1e-5))
    s1 = p["g1"] * c
    w1 = p["lin1"]["w"]
    v0 = s1 * p["lin1"]["b"] + p["be1"]
    s2 = p["g2"] * c
    return s1, w1, v0, p["lin2"]["w"].T, p["lin2"]["b"], s2, p["be2"]


def _edge_perm():
    r = np.arange(EB // 4)
    evens = np.stack([4 * r, 4 * r + 1], 1).reshape(-1)
    odds = np.stack([4 * r + 2, 4 * r + 3], 1).reshape(-1)
    block_order = np.concatenate([evens, odds])
    return (np.arange(E_PAD // EB)[:, None] * EB
            + block_order[None, :]).reshape(-1)


_EDGE_PERM = _edge_perm()


def kernel(x, edge_attr, params, edge_index, batch):
    pad = E_PAD - E
    dst1 = jnp.concatenate([edge_index[1], jnp.zeros((pad,), jnp.int32)])
    src1 = jnp.concatenate([edge_index[0], jnp.zeros((pad,), jnp.int32)])
    dstp = dst1.reshape(E_PAD // 128, 128)
    srcp = src1.reshape(E_PAD // 128, 128)
    eap = jnp.concatenate([edge_attr.reshape(-1),
                           jnp.zeros((pad,), jnp.float32)])
    ea4 = eap.reshape(E_PAD // 4, 4)
    dperm = dst1[jnp.asarray(_EDGE_PERM)]
    dst2 = (2 * dperm[:, None]
            + jnp.arange(2, dtype=jnp.int32)[None, :]).reshape(HR // 128, 128)

    layers = []
    for cp in params["convs"]:
        s1, w1, v0, w2T, b2, s2, be2 = _fold_mlp(cp["msg"])
        wdT = (w1[:, :HID] * s1[:, None]).T
        wsT = (w1[:, HID:2 * HID] * s1[:, None]).T
        v1 = s1 * w1[:, 2 * HID]
        kmat = jnp.zeros((4, 128), jnp.float32)
        for i in range(4):
            kmat = kmat.at[i, 32 * i:32 * (i + 1)].set(v1)
        we = jnp.zeros((128, 128), jnp.float32)
        we = we.at[0:32, 0:64].set(w2T).at[32:64, 64:128].set(w2T)
        wo = jnp.zeros((128, 128), jnp.float32)
        wo = wo.at[64:96, 0:64].set(w2T).at[96:128, 64:128].set(w2T)
        s1u, u1, v0u, u2T, b2u, s2u, be2u = _fold_mlp(cp["upd"])
        uhT = (u1[:, :HID] * s1u[:, None]).T
        uaT = (u1[:, HID:] * s1u[:, None]).T
        two = lambda v: jnp.concatenate([v, v]).reshape(1, 128)
        layers.append(dict(
            wdT=wdT, wsT=wsT, v0=v0.reshape(1, HL), kmat=kmat, we=we, wo=wo,
            b2=two(b2), s2=two(s2), be2=two(be2),
            uhT=uhT, uaT=uaT, v0u=v0u.reshape(1, HL), u2T=u2T,
            b2u=b2u.reshape(1, HID), s2u=s2u.reshape(1, HID),
            be2u=be2u.reshape(1, HID)))

    winT = params["lin_in"]["w"].T
    bin_ = params["lin_in"]["b"].reshape(1, HID)
    woutT = params["lin_out"]["w"].T
    bout = params["lin_out"]["b"].reshape(1, 1)

    f32 = jnp.float32
    h, a, b = pl.pallas_call(
        _k1_body,
        out_shape=[jax.ShapeDtypeStruct((N, HID), f32),
                   jax.ShapeDtypeStruct((N, HL), f32),
                   jax.ShapeDtypeStruct((N, HL), f32)],
    )(x, winT, bin_, layers[0]["wdT"], layers[0]["wsT"], layers[0]["v0"])

    s1_gather, s2_scatter = _sc_kernels()
    for li, ly in enumerate(layers):
        pre4 = s1_gather(a, b, dstp, srcp)
        m2 = pl.pallas_call(
            _k2_body,
            grid=(E_PAD // EB,),
            in_specs=[
                pl.BlockSpec((EB // 4, 128), lambda i: (i, 0)),
                pl.BlockSpec((EB // 4, 4), lambda i: (i, 0)),
                pl.BlockSpec((4, 128), lambda i: (0, 0)),
                pl.BlockSpec((128, 128), lambda i: (0, 0)),
                pl.BlockSpec((128, 128), lambda i: (0, 0)),
                pl.BlockSpec((1, 128), lambda i: (0, 0)),
                pl.BlockSpec((1, 128), lambda i: (0, 0)),
                pl.BlockSpec((1, 128), lambda i: (0, 0)),
            ],
            out_specs=pl.BlockSpec((EB // 2, 128), lambda i: (i, 0)),
            out_shape=jax.ShapeDtypeStruct((E_PAD // 2, 128), f32),
        )(pre4, ea4, ly["kmat"], ly["we"], ly["wo"], ly["b2"], ly["s2"],
          ly["be2"])
        ag = s2_scatter(m2.reshape(HR, HL), dst2)
        nxt = layers[li + 1] if li + 1 < len(layers) else layers[0]
        h, a, b = pl.pallas_call(
            _k3_body,
            out_shape=[jax.ShapeDtypeStruct((N, HID), f32),
                       jax.ShapeDtypeStruct((N, HL), f32),
                       jax.ShapeDtypeStruct((N, HL), f32)],
        )(h, ag.reshape(NC, N_PAD, HID), ly["uhT"], ly["uaT"], ly["v0u"],
          ly["u2T"], ly["b2u"], ly["s2u"], ly["be2u"],
          nxt["wdT"], nxt["wsT"], nxt["v0"])

    out = pl.pallas_call(
        _k4_body,
        out_shape=jax.ShapeDtypeStruct((G, 1), f32),
    )(h, batch.reshape(1, N), woutT, bout)
    return out.reshape(-1)


# speedup vs baseline: 8.3511x; 1.2008x over previous
import functools

import jax
import jax.numpy as jnp
import numpy as np
from jax import lax
from jax.experimental import pallas as pl
from jax.experimental.pallas import tpu as pltpu
from jax.experimental.pallas import tpu_sc as plsc

N = 10000
E = 320000
IN_DIM = 128
HID = 64
HL = 32
G = 64

NC = 2
NS = 16
NW = NC * NS

EW = 10240
E_PAD = NW * EW
SCH = 512
NSCH = EW // SCH
HR = 2 * E_PAD
HRW = HR // NW
SCH2 = 1024
NSCH2 = HRW // SCH2
N_PAD = 10240
RPT = N_PAD // NS
EB = 16384


def _s1_gather_body(a_hbm, b_hbm, dst_hbm, src_hbm, pre_hbm,
                    idxd, idxs,
                    bufa0, bufb0, bufa1, bufb1, bufo0, bufo1,
                    sgi, sga0, sgb0, sga1, sgb1, so0, so1):
    wid = lax.axis_index("s") * NC + lax.axis_index("c")
    base_w = wid * EW
    bufa = [bufa0, bufa1]
    bufb = [bufb0, bufb1]
    bufo = [bufo0, bufo1]
    sga = [sga0, sga1]
    sgb = [sgb0, sgb1]
    so = [so0, so1]

    rb_w = pl.multiple_of(base_w // 128, 8)
    cp1 = pltpu.async_copy(dst_hbm.at[pl.ds(rb_w, EW // 128)], idxd, sgi)
    cp2 = pltpu.async_copy(src_hbm.at[pl.ds(rb_w, EW // 128)], idxs, sgi)
    cp1.wait()
    cp2.wait()

    def fetch(i, s):
        rofs = i * (SCH // 128)
        for j in range(SCH // 128):
            pltpu.async_copy(a_hbm.at[idxd.at[rofs + j]],
                             bufa[s].at[pl.ds(j * 128, 128)], sga[s])
            pltpu.async_copy(b_hbm.at[idxs.at[rofs + j]],
                             bufb[s].at[pl.ds(j * 128, 128)], sgb[s])

    def wait_fetch(s):
        pltpu.make_async_copy(a_hbm.at[pl.ds(0, SCH)], bufa[s], sga[s]).wait()
        pltpu.make_async_copy(b_hbm.at[pl.ds(0, SCH)], bufb[s], sgb[s]).wait()

    def combine(s):
        ba, bb, bo = bufa[s], bufb[s], bufo[s]

        def group(g, _):
            j0 = g * 16
            r0 = g * 4
            for i in range(16):
                j = j0 + i
                c = (i % 4) * 32
                r = r0 + i // 4
                t = ba[j, pl.ds(0, 32)] + bb[j, pl.ds(0, 32)]
                lo, hi = plsc.unpack(t, format=plsc.PackFormat.INTERLEAVED)
                bo[r, pl.ds(c, 16)] = lo
                bo[r, pl.ds(c + 16, 16)] = hi
            return 0

        lax.fori_loop(0, SCH // 16, group, 0)

    def put(i, s):
        base = pl.multiple_of(base_w + i * SCH, SCH)
        pltpu.async_copy(bufo[s], pre_hbm.at[pl.ds(base // 4, SCH // 4)],
                         so[s])

    def drain_put(s):
        pltpu.make_async_copy(pre_hbm.at[pl.ds(0, SCH // 4)],
                              bufo[s], so[s]).wait()

    fetch(0, 0)

    def pair(ii, _):
        i = 2 * ii
        fetch(i + 1, 1)
        wait_fetch(0)
        combine(0)

        @pl.when(ii > 0)
        def _():
            drain_put(0)

        put(i, 0)

        @pl.when(ii + 1 < NSCH // 2)
        def _():
            fetch(i + 2, 0)

        wait_fetch(1)
        combine(1)

        @pl.when(ii > 0)
        def _():
            drain_put(1)

        put(i + 1, 1)
        return 0

    lax.fori_loop(0, NSCH // 2, pair, 0)
    drain_put(0)
    drain_put(1)


def _s2_scatter_body(m_hbm, dst_hbm, out_hbm,
                     idx, mbuf0, mbuf1, zbuf, acc,
                     sgi, sm0, sm1, ssc):
    cid = lax.axis_index("c")
    sid = lax.axis_index("s")
    wid = cid * NS + sid
    base_w = wid * HRW
    mbuf = [mbuf0, mbuf1]
    sm = [sm0, sm1]

    rb_w = pl.multiple_of(base_w // 128, 8)
    pltpu.async_copy(dst_hbm.at[pl.ds(rb_w, HRW // 128)], idx, sgi).wait()

    def zrow(j, _):
        zbuf[j, pl.ds(0, 16)] = jnp.zeros((16,), jnp.float32)
        zbuf[j, pl.ds(16, 16)] = jnp.zeros((16,), jnp.float32)
        return 0

    lax.fori_loop(0, 64, zrow, 0)

    def zcopy(k, _):
        pltpu.sync_copy(
            zbuf,
            acc.at[pl.ds(pl.multiple_of(sid * 2 * RPT + k * 64, 64), 64)])
        return 0

    lax.fori_loop(0, 2 * RPT // 64, zcopy, 0)
    plsc.subcore_barrier()

    def fetch(i, s):
        base = pl.multiple_of(base_w + i * SCH2, SCH2)
        pltpu.async_copy(m_hbm.at[pl.ds(base, SCH2)], mbuf[s], sm[s])

    def wait_fetch(s):
        pltpu.make_async_copy(m_hbm.at[pl.ds(0, SCH2)], mbuf[s],
                              sm[s]).wait()

    def scatter(i, s):
        rofs = i * (SCH2 // 128)
        for j in range(SCH2 // 128):
            pltpu.async_copy(mbuf[s].at[pl.ds(j * 128, 128)],
                             acc.at[idx.at[rofs + j]], ssc, add=True)
        pltpu.make_async_copy(m_hbm.at[pl.ds(0, SCH2)], mbuf[s],
                              ssc).wait()

    fetch(0, 0)

    def pair(ii, _):
        i = 2 * ii
        fetch(i + 1, 1)
        wait_fetch(0)
        scatter(i, 0)

        @pl.when(ii + 1 < NSCH2 // 2)
        def _():
            fetch(i + 2, 0)

        wait_fetch(1)
        scatter(i + 1, 1)
        return 0

    lax.fori_loop(0, NSCH2 // 2, pair, 0)
    plsc.subcore_barrier()
    srow = pl.multiple_of(sid * 2 * RPT, 2 * RPT)
    pltpu.sync_copy(acc.at[pl.ds(srow, 2 * RPT)],
                    out_hbm.at[cid, pl.ds(srow, 2 * RPT)])


@functools.cache
def _sc_kernels():
    mesh = plsc.VectorSubcoreMesh(
        core_axis_name="c", subcore_axis_name="s",
        num_cores=NC, num_subcores=NS)
    scp = pltpu.CompilerParams(use_tc_tiling_on_sc=False, needs_layout_passes=False)
    i32, f32 = jnp.int32, jnp.float32
    s1 = pl.kernel(
        _s1_gather_body,
        out_type=jax.ShapeDtypeStruct((E_PAD // 4, 128), f32),
        mesh=mesh,
        compiler_params=scp,
        scratch_types=(
            [pltpu.VMEM((EW // 128, 128), i32) for _ in range(2)]
            + [pltpu.VMEM((SCH, HL), jnp.bfloat16) for _ in range(4)]
            + [pltpu.VMEM((SCH // 4, 128), f32) for _ in range(2)]
            + [pltpu.SemaphoreType.DMA for _ in range(7)]
        ))
    s2 = pl.kernel(
        _s2_scatter_body,
        out_type=jax.ShapeDtypeStruct((NC, 2 * N_PAD, HL), f32),
        mesh=mesh,
        compiler_params=scp,
        scratch_types=(
            [pltpu.VMEM((HRW // 128, 128), i32)]
            + [pltpu.VMEM((SCH2, HL), f32) for _ in range(2)]
            + [pltpu.VMEM((64, HL), f32)]
            + [pltpu.VMEM_SHARED((2 * N_PAD, HL), f32)]
            + [pltpu.SemaphoreType.DMA for _ in range(4)]
        ))
    return s1, s2


def _k1_body(x_ref, winT, bin_, wdT, wsT, v0, h_ref, a_ref, b_ref):
    h = jnp.dot(x_ref[...], winT[...],
                preferred_element_type=jnp.float32) + bin_[...]
    h_ref[...] = h
    a_ref[...] = (jnp.dot(h, wdT[...], preferred_element_type=jnp.float32)
                  + v0[...]).astype(jnp.bfloat16)
    b_ref[...] = jnp.dot(
        h, wsT[...], preferred_element_type=jnp.float32).astype(jnp.bfloat16)


def _k2_body(pre_ref, ea_ref, kmat, we, wo, b2_, s2_, be2_, m_ref):
    pid = pl.program_id(0)
    eterm = lax.dot_general(
        ea_ref[...], kmat[...], (((0,), (0,)), ((), ())),
        preferred_element_type=jnp.float32)
    z = jax.nn.relu(pre_ref[...] + eterm)
    rows = pid * EB + 4 * lax.broadcasted_iota(jnp.int32, (EB // 4, 1), 0)
    valid = rows < E
    meven = s2_[...] * jax.nn.relu(
        jnp.dot(z, we[...], preferred_element_type=jnp.float32)
        + b2_[...]) + be2_[...]
    modd = s2_[...] * jax.nn.relu(
        jnp.dot(z, wo[...], preferred_element_type=jnp.float32)
        + b2_[...]) + be2_[...]
    m_ref[:EB // 4, :] = jnp.where(valid, meven, 0.0)
    m_ref[EB // 4:, :] = jnp.where(valid, modd, 0.0)


def _k3_body(h_ref, ag_ref, uhT, uaT, v0u, u2T, b2u, s2u, be2u,
             wdT, wsT, v0n, hn_ref, a_ref, b_ref):
    h = h_ref[...]
    aggr = ag_ref[0, :N, :] + ag_ref[1, :N, :]
    t = jax.nn.relu(jnp.dot(h, uhT[...], preferred_element_type=jnp.float32)
                    + jnp.dot(aggr, uaT[...], preferred_element_type=jnp.float32)
                    + v0u[...])
    u2 = jnp.dot(t, u2T[...], preferred_element_type=jnp.float32) + b2u[...]
    hn = h + s2u[...] * jax.nn.relu(u2) + be2u[...]
    hn_ref[...] = hn
    a_ref[...] = (jnp.dot(hn, wdT[...], preferred_element_type=jnp.float32)
                  + v0n[...]).astype(jnp.bfloat16)
    b_ref[...] = jnp.dot(
        hn, wsT[...], preferred_element_type=jnp.float32).astype(jnp.bfloat16)


def _k4_body(h_ref, batch_ref, woutT, bout, out_ref):
    b = batch_ref[...]
    mask = (b == lax.broadcasted_iota(jnp.int32, (G, N), 0)).astype(jnp.float32)
    sums = jnp.dot(mask, h_ref[...], preferred_element_type=jnp.float32)
    counts = jnp.sum(mask, axis=1, keepdims=True)
    pooled = sums / jnp.maximum(counts, 1.0)
    out = jnp.dot(pooled, woutT[...], preferred_element_type=jnp.float32) + bout[...]
    out_ref[...] = jax.nn.relu(out)


def _fold_mlp(p):
    c = 1.0 / jnp.sqrt(jnp.float32(1.0 + 1e-5))
    s1 = p["g1"] * c
    w1 = p["lin1"]["w"]
    v0 = s1 * p["lin1"]["b"] + p["be1"]
    s2 = p["g2"] * c
    return s1, w1, v0, p["lin2"]["w"].T, p["lin2"]["b"], s2, p["be2"]


def _edge_perm():
    r = np.arange(EB // 4)
    evens = np.stack([4 * r, 4 * r + 1], 1).reshape(-1)
    odds = np.stack([4 * r + 2, 4 * r + 3], 1).reshape(-1)
    block_order = np.concatenate([evens, odds])
    return (np.arange(E_PAD // EB)[:, None] * EB
            + block_order[None, :]).reshape(-1)


_EDGE_PERM = _edge_perm()

_SWZ = np.stack([np.arange(16), np.arange(16) + 16], 1).reshape(-1)


def kernel(x, edge_attr, params, edge_index, batch):
    pad = E_PAD - E
    dst1 = jnp.concatenate([edge_index[1], jnp.zeros((pad,), jnp.int32)])
    src1 = jnp.concatenate([edge_index[0], jnp.zeros((pad,), jnp.int32)])
    dstp = dst1.reshape(E_PAD // 128, 128)
    srcp = src1.reshape(E_PAD // 128, 128)
    eap = jnp.concatenate([edge_attr.reshape(-1),
                           jnp.zeros((pad,), jnp.float32)])
    ea4 = eap.reshape(E_PAD // 4, 4).T
    dperm = dst1[jnp.asarray(_EDGE_PERM)]
    dst2 = (2 * dperm[:, None]
            + jnp.arange(2, dtype=jnp.int32)[None, :]).reshape(HR // 128, 128)

    layers = []
    for cp in params["convs"]:
        s1, w1, v0, w2T, b2, s2, be2 = _fold_mlp(cp["msg"])
        wdT = (w1[:, :HID] * s1[:, None]).T[:, _SWZ]
        wsT = (w1[:, HID:2 * HID] * s1[:, None]).T[:, _SWZ]
        v1 = s1 * w1[:, 2 * HID]
        kmat = jnp.zeros((4, 128), jnp.float32)
        for i in range(4):
            kmat = kmat.at[i, 32 * i:32 * (i + 1)].set(v1)
        we = jnp.zeros((128, 128), jnp.float32)
        we = we.at[0:32, 0:64].set(w2T).at[32:64, 64:128].set(w2T)
        wo = jnp.zeros((128, 128), jnp.float32)
        wo = wo.at[64:96, 0:64].set(w2T).at[96:128, 64:128].set(w2T)
        s1u, u1, v0u, u2T, b2u, s2u, be2u = _fold_mlp(cp["upd"])
        uhT = (u1[:, :HID] * s1u[:, None]).T
        uaT = (u1[:, HID:] * s1u[:, None]).T
        two = lambda v: jnp.concatenate([v, v]).reshape(1, 128)
        layers.append(dict(
            wdT=wdT, wsT=wsT, v0=v0[_SWZ].reshape(1, HL), kmat=kmat,
            we=we, wo=wo,
            b2=two(b2), s2=two(s2), be2=two(be2),
            uhT=uhT, uaT=uaT, v0u=v0u.reshape(1, HL), u2T=u2T,
            b2u=b2u.reshape(1, HID), s2u=s2u.reshape(1, HID),
            be2u=be2u.reshape(1, HID)))

    winT = params["lin_in"]["w"].T
    bin_ = params["lin_in"]["b"].reshape(1, HID)
    woutT = params["lin_out"]["w"].T
    bout = params["lin_out"]["b"].reshape(1, 1)

    f32 = jnp.float32
    h, a, b = pl.pallas_call(
        _k1_body,
        out_shape=[jax.ShapeDtypeStruct((N, HID), f32),
                   jax.ShapeDtypeStruct((N, HL), jnp.bfloat16),
                   jax.ShapeDtypeStruct((N, HL), jnp.bfloat16)],
    )(x, winT, bin_, layers[0]["wdT"], layers[0]["wsT"], layers[0]["v0"])

    s1_gather, s2_scatter = _sc_kernels()
    for li, ly in enumerate(layers):
        pre4 = s1_gather(a, b, dstp, srcp)
        m2 = pl.pallas_call(
            _k2_body,
            grid=(E_PAD // EB,),
            in_specs=[
                pl.BlockSpec((EB // 4, 128), lambda i: (i, 0)),
                pl.BlockSpec((4, EB // 4), lambda i: (0, i)),
                pl.BlockSpec((4, 128), lambda i: (0, 0)),
                pl.BlockSpec((128, 128), lambda i: (0, 0)),
                pl.BlockSpec((128, 128), lambda i: (0, 0)),
                pl.BlockSpec((1, 128), lambda i: (0, 0)),
                pl.BlockSpec((1, 128), lambda i: (0, 0)),
                pl.BlockSpec((1, 128), lambda i: (0, 0)),
            ],
            out_specs=pl.BlockSpec((EB // 2, 128), lambda i: (i, 0)),
            out_shape=jax.ShapeDtypeStruct((E_PAD // 2, 128), f32),
        )(pre4, ea4, ly["kmat"], ly["we"], ly["wo"], ly["b2"], ly["s2"],
          ly["be2"])
        ag = s2_scatter(m2.reshape(HR, HL), dst2)
        nxt = layers[li + 1] if li + 1 < len(layers) else layers[0]
        h, a, b = pl.pallas_call(
            _k3_body,
            out_shape=[jax.ShapeDtypeStruct((N, HID), f32),
                       jax.ShapeDtypeStruct((N, HL), jnp.bfloat16),
                       jax.ShapeDtypeStruct((N, HL), jnp.bfloat16)],
        )(h, ag.reshape(NC, N_PAD, HID), ly["uhT"], ly["uaT"], ly["v0u"],
          ly["u2T"], ly["b2u"], ly["s2u"], ly["be2u"],
          nxt["wdT"], nxt["wsT"], nxt["v0"])

    out = pl.pallas_call(
        _k4_body,
        out_shape=jax.ShapeDtypeStruct((G, 1), f32),
    )(h, batch.reshape(1, N), woutT, bout)
    return out.reshape(-1)


# speedup vs baseline: 9.2954x; 1.1131x over previous
import functools

import jax
import jax.numpy as jnp
import numpy as np
from jax import lax
from jax.experimental import pallas as pl
from jax.experimental.pallas import tpu as pltpu
from jax.experimental.pallas import tpu_sc as plsc

N = 10000
E = 320000
IN_DIM = 128
HID = 64
HL = 32
G = 64

NC = 2
NS = 16
NW = NC * NS

EW = 10240
E_PAD = NW * EW
SCH = 512
NSCH = EW // SCH
HR = 2 * E_PAD
HRW = HR // NW
SCH2 = 1024
NSCH2 = HRW // SCH2
N_PAD = 10240
RPT = N_PAD // NS
EB = 16384


def _s1_gather_body(a_hbm, b_hbm, dst_hbm, src_hbm, pre_hbm,
                    idxd, idxs,
                    bufa0, bufb0, bufa1, bufb1, bufo0, bufo1,
                    taba, tabb,
                    sgi, sga0, sgb0, sga1, sgb1, so0, so1):
    sid = lax.axis_index("s")
    wid = sid * NC + lax.axis_index("c")
    base_w = wid * EW
    tr0 = pl.multiple_of(sid * 625, 25)
    pltpu.async_copy(a_hbm.at[pl.ds(tr0, 625)], taba.at[pl.ds(tr0, 625)],
                     sgi)
    pltpu.async_copy(b_hbm.at[pl.ds(tr0, 625)], tabb.at[pl.ds(tr0, 625)],
                     sgi)
    bufa = [bufa0, bufa1]
    bufb = [bufb0, bufb1]
    bufo = [bufo0, bufo1]
    sga = [sga0, sga1]
    sgb = [sgb0, sgb1]
    so = [so0, so1]

    rb_w = pl.multiple_of(base_w // 128, 8)
    cp1 = pltpu.async_copy(dst_hbm.at[pl.ds(rb_w, EW // 128)], idxd, sgi)
    cp2 = pltpu.async_copy(src_hbm.at[pl.ds(rb_w, EW // 128)], idxs, sgi)
    pltpu.make_async_copy(a_hbm.at[pl.ds(0, 625)],
                          taba.at[pl.ds(0, 625)], sgi).wait()
    pltpu.make_async_copy(b_hbm.at[pl.ds(0, 625)],
                          tabb.at[pl.ds(0, 625)], sgi).wait()
    cp1.wait()
    cp2.wait()
    plsc.subcore_barrier()

    def fetch(i, s):
        rofs = i * (SCH // 128)
        for j in range(SCH // 128):
            pltpu.async_copy(taba.at[idxd.at[rofs + j]],
                             bufa[s].at[pl.ds(j * 128, 128)], sga[s])
            pltpu.async_copy(tabb.at[idxs.at[rofs + j]],
                             bufb[s].at[pl.ds(j * 128, 128)], sgb[s])

    def wait_fetch(s):
        pltpu.make_async_copy(a_hbm.at[pl.ds(0, SCH)], bufa[s], sga[s]).wait()
        pltpu.make_async_copy(b_hbm.at[pl.ds(0, SCH)], bufb[s], sgb[s]).wait()

    def combine(s):
        ba, bb, bo = bufa[s], bufb[s], bufo[s]

        def group(g, _):
            j0 = g * 16
            r0 = g * 4
            for i in range(16):
                j = j0 + i
                c = (i % 4) * 32
                r = r0 + i // 4
                t = ba[j, pl.ds(0, 32)] + bb[j, pl.ds(0, 32)]
                lo, hi = plsc.unpack(t, format=plsc.PackFormat.INTERLEAVED)
                bo[r, pl.ds(c, 16)] = lo
                bo[r, pl.ds(c + 16, 16)] = hi
            return 0

        lax.fori_loop(0, SCH // 16, group, 0)

    def put(i, s):
        base = pl.multiple_of(base_w + i * SCH, SCH)
        pltpu.async_copy(bufo[s], pre_hbm.at[pl.ds(base // 4, SCH // 4)],
                         so[s])

    def drain_put(s):
        pltpu.make_async_copy(pre_hbm.at[pl.ds(0, SCH // 4)],
                              bufo[s], so[s]).wait()

    fetch(0, 0)

    def pair(ii, _):
        i = 2 * ii
        fetch(i + 1, 1)
        wait_fetch(0)
        combine(0)

        @pl.when(ii > 0)
        def _():
            drain_put(0)

        put(i, 0)

        @pl.when(ii + 1 < NSCH // 2)
        def _():
            fetch(i + 2, 0)

        wait_fetch(1)
        combine(1)

        @pl.when(ii > 0)
        def _():
            drain_put(1)

        put(i + 1, 1)
        return 0

    lax.fori_loop(0, NSCH // 2, pair, 0)
    drain_put(0)
    drain_put(1)


def _s2_scatter_body(m_hbm, dst_hbm, out_hbm,
                     idx, mbuf0, mbuf1, zbuf, acc,
                     sgi, sm0, sm1, ssc):
    cid = lax.axis_index("c")
    sid = lax.axis_index("s")
    wid = cid * NS + sid
    base_w = wid * HRW
    mbuf = [mbuf0, mbuf1]
    sm = [sm0, sm1]

    rb_w = pl.multiple_of(base_w // 128, 8)
    pltpu.async_copy(dst_hbm.at[pl.ds(rb_w, HRW // 128)], idx, sgi).wait()

    def zrow(j, _):
        zbuf[j, pl.ds(0, 16)] = jnp.zeros((16,), jnp.float32)
        zbuf[j, pl.ds(16, 16)] = jnp.zeros((16,), jnp.float32)
        return 0

    lax.fori_loop(0, 64, zrow, 0)

    def zcopy(k, _):
        pltpu.sync_copy(
            zbuf,
            acc.at[pl.ds(pl.multiple_of(sid * 2 * RPT + k * 64, 64), 64)])
        return 0

    lax.fori_loop(0, 2 * RPT // 64, zcopy, 0)
    plsc.subcore_barrier()

    def fetch(i, s):
        base = pl.multiple_of(base_w + i * SCH2, SCH2)
        pltpu.async_copy(m_hbm.at[pl.ds(base, SCH2)], mbuf[s], sm[s])

    def wait_fetch(s):
        pltpu.make_async_copy(m_hbm.at[pl.ds(0, SCH2)], mbuf[s],
                              sm[s]).wait()

    def scatter(i, s):
        rofs = i * (SCH2 // 128)
        for j in range(SCH2 // 128):
            pltpu.async_copy(mbuf[s].at[pl.ds(j * 128, 128)],
                             acc.at[idx.at[rofs + j]], ssc, add=True)
        pltpu.make_async_copy(m_hbm.at[pl.ds(0, SCH2)], mbuf[s],
                              ssc).wait()

    fetch(0, 0)

    def pair(ii, _):
        i = 2 * ii
        fetch(i + 1, 1)
        wait_fetch(0)
        scatter(i, 0)

        @pl.when(ii + 1 < NSCH2 // 2)
        def _():
            fetch(i + 2, 0)

        wait_fetch(1)
        scatter(i + 1, 1)
        return 0

    lax.fori_loop(0, NSCH2 // 2, pair, 0)
    plsc.subcore_barrier()
    srow = pl.multiple_of(sid * 2 * RPT, 2 * RPT)
    pltpu.sync_copy(acc.at[pl.ds(srow, 2 * RPT)],
                    out_hbm.at[cid, pl.ds(srow, 2 * RPT)])


@functools.cache
def _sc_kernels():
    mesh = plsc.VectorSubcoreMesh(
        core_axis_name="c", subcore_axis_name="s",
        num_cores=NC, num_subcores=NS)
    scp = pltpu.CompilerParams(use_tc_tiling_on_sc=False, needs_layout_passes=False)
    i32, f32 = jnp.int32, jnp.float32
    s1 = pl.kernel(
        _s1_gather_body,
        out_type=jax.ShapeDtypeStruct((E_PAD // 4, 128), f32),
        mesh=mesh,
        compiler_params=scp,
        scratch_types=(
            [pltpu.VMEM((EW // 128, 128), i32) for _ in range(2)]
            + [pltpu.VMEM((SCH, HL), jnp.bfloat16) for _ in range(4)]
            + [pltpu.VMEM((SCH // 4, 128), f32) for _ in range(2)]
            + [pltpu.VMEM_SHARED((N, HL), jnp.bfloat16) for _ in range(2)]
            + [pltpu.SemaphoreType.DMA for _ in range(7)]
        ))
    s2 = pl.kernel(
        _s2_scatter_body,
        out_type=jax.ShapeDtypeStruct((NC, 2 * N_PAD, HL), f32),
        mesh=mesh,
        compiler_params=scp,
        scratch_types=(
            [pltpu.VMEM((HRW // 128, 128), i32)]
            + [pltpu.VMEM((SCH2, HL), f32) for _ in range(2)]
            + [pltpu.VMEM((64, HL), f32)]
            + [pltpu.VMEM_SHARED((2 * N_PAD, HL), f32)]
            + [pltpu.SemaphoreType.DMA for _ in range(4)]
        ))
    return s1, s2


def _k1_body(x_ref, winT, bin_, wdT, wsT, v0, h_ref, a_ref, b_ref):
    h = jnp.dot(x_ref[...], winT[...],
                preferred_element_type=jnp.float32) + bin_[...]
    h_ref[...] = h
    a_ref[...] = (jnp.dot(h, wdT[...], preferred_element_type=jnp.float32)
                  + v0[...]).astype(jnp.bfloat16)
    b_ref[...] = jnp.dot(
        h, wsT[...], preferred_element_type=jnp.float32).astype(jnp.bfloat16)


def _k2_body(pre_ref, ea_ref, kmat, we, wo, b2_, s2_, be2_, m_ref):
    pid = pl.program_id(0)
    eterm = lax.dot_general(
        ea_ref[...], kmat[...], (((0,), (0,)), ((), ())),
        preferred_element_type=jnp.float32)
    z = jax.nn.relu(pre_ref[...] + eterm)
    rows = pid * EB + 4 * lax.broadcasted_iota(jnp.int32, (EB // 4, 1), 0)
    valid = rows < E
    meven = s2_[...] * jax.nn.relu(
        jnp.dot(z, we[...], preferred_element_type=jnp.float32)
        + b2_[...]) + be2_[...]
    modd = s2_[...] * jax.nn.relu(
        jnp.dot(z, wo[...], preferred_element_type=jnp.float32)
        + b2_[...]) + be2_[...]
    m_ref[:EB // 4, :] = jnp.where(valid, meven, 0.0)
    m_ref[EB // 4:, :] = jnp.where(valid, modd, 0.0)


def _k3_body(h_ref, ag_ref, uhT, uaT, v0u, u2T, b2u, s2u, be2u,
             wdT, wsT, v0n, hn_ref, a_ref, b_ref):
    h = h_ref[...]
    aggr = ag_ref[0, :N, :] + ag_ref[1, :N, :]
    t = jax.nn.relu(jnp.dot(h, uhT[...], preferred_element_type=jnp.float32)
                    + jnp.dot(aggr, uaT[...], preferred_element_type=jnp.float32)
                    + v0u[...])
    u2 = jnp.dot(t, u2T[...], preferred_element_type=jnp.float32) + b2u[...]
    hn = h + s2u[...] * jax.nn.relu(u2) + be2u[...]
    hn_ref[...] = hn
    a_ref[...] = (jnp.dot(hn, wdT[...], preferred_element_type=jnp.float32)
                  + v0n[...]).astype(jnp.bfloat16)
    b_ref[...] = jnp.dot(
        hn, wsT[...], preferred_element_type=jnp.float32).astype(jnp.bfloat16)


def _k4_body(h_ref, batch_ref, woutT, bout, out_ref):
    b = batch_ref[...]
    mask = (b == lax.broadcasted_iota(jnp.int32, (G, N), 0)).astype(jnp.float32)
    sums = jnp.dot(mask, h_ref[...], preferred_element_type=jnp.float32)
    counts = jnp.sum(mask, axis=1, keepdims=True)
    pooled = sums / jnp.maximum(counts, 1.0)
    out = jnp.dot(pooled, woutT[...], preferred_element_type=jnp.float32) + bout[...]
    out_ref[...] = jax.nn.relu(out)


def _fold_mlp(p):
    c = 1.0 / jnp.sqrt(jnp.float32(1.0 + 1e-5))
    s1 = p["g1"] * c
    w1 = p["lin1"]["w"]
    v0 = s1 * p["lin1"]["b"] + p["be1"]
    s2 = p["g2"] * c
    return s1, w1, v0, p["lin2"]["w"].T, p["lin2"]["b"], s2, p["be2"]


def _edge_perm():
    r = np.arange(EB // 4)
    evens = np.stack([4 * r, 4 * r + 1], 1).reshape(-1)
    odds = np.stack([4 * r + 2, 4 * r + 3], 1).reshape(-1)
    block_order = np.concatenate([evens, odds])
    return (np.arange(E_PAD // EB)[:, None] * EB
            + block_order[None, :]).reshape(-1)


_EDGE_PERM = _edge_perm()

_SWZ = np.stack([np.arange(16), np.arange(16) + 16], 1).reshape(-1)


def kernel(x, edge_attr, params, edge_index, batch):
    pad = E_PAD - E
    dst1 = jnp.concatenate([edge_index[1], jnp.zeros((pad,), jnp.int32)])
    src1 = jnp.concatenate([edge_index[0], jnp.zeros((pad,), jnp.int32)])
    dstp = dst1.reshape(E_PAD // 128, 128)
    srcp = src1.reshape(E_PAD // 128, 128)
    eap = jnp.concatenate([edge_attr.reshape(-1),
                           jnp.zeros((pad,), jnp.float32)])
    ea4 = eap.reshape(E_PAD // 4, 4).T
    dperm = dst1[jnp.asarray(_EDGE_PERM)]
    dst2 = (2 * dperm[:, None]
            + jnp.arange(2, dtype=jnp.int32)[None, :]).reshape(HR // 128, 128)

    layers = []
    for cp in params["convs"]:
        s1, w1, v0, w2T, b2, s2, be2 = _fold_mlp(cp["msg"])
        wdT = (w1[:, :HID] * s1[:, None]).T[:, _SWZ]
        wsT = (w1[:, HID:2 * HID] * s1[:, None]).T[:, _SWZ]
        v1 = s1 * w1[:, 2 * HID]
        kmat = jnp.zeros((4, 128), jnp.float32)
        for i in range(4):
            kmat = kmat.at[i, 32 * i:32 * (i + 1)].set(v1)
        we = jnp.zeros((128, 128), jnp.float32)
        we = we.at[0:32, 0:64].set(w2T).at[32:64, 64:128].set(w2T)
        wo = jnp.zeros((128, 128), jnp.float32)
        wo = wo.at[64:96, 0:64].set(w2T).at[96:128, 64:128].set(w2T)
        s1u, u1, v0u, u2T, b2u, s2u, be2u = _fold_mlp(cp["upd"])
        uhT = (u1[:, :HID] * s1u[:, None]).T
        uaT = (u1[:, HID:] * s1u[:, None]).T
        two = lambda v: jnp.concatenate([v, v]).reshape(1, 128)
        layers.append(dict(
            wdT=wdT, wsT=wsT, v0=v0[_SWZ].reshape(1, HL), kmat=kmat,
            we=we, wo=wo,
            b2=two(b2), s2=two(s2), be2=two(be2),
            uhT=uhT, uaT=uaT, v0u=v0u.reshape(1, HL), u2T=u2T,
            b2u=b2u.reshape(1, HID), s2u=s2u.reshape(1, HID),
            be2u=be2u.reshape(1, HID)))

    winT = params["lin_in"]["w"].T
    bin_ = params["lin_in"]["b"].reshape(1, HID)
    woutT = params["lin_out"]["w"].T
    bout = params["lin_out"]["b"].reshape(1, 1)

    f32 = jnp.float32
    h, a, b = pl.pallas_call(
        _k1_body,
        out_shape=[jax.ShapeDtypeStruct((N, HID), f32),
                   jax.ShapeDtypeStruct((N, HL), jnp.bfloat16),
                   jax.ShapeDtypeStruct((N, HL), jnp.bfloat16)],
    )(x, winT, bin_, layers[0]["wdT"], layers[0]["wsT"], layers[0]["v0"])

    s1_gather, s2_scatter = _sc_kernels()
    for li, ly in enumerate(layers):
        pre4 = s1_gather(a, b, dstp, srcp)
        m2 = pl.pallas_call(
            _k2_body,
            grid=(E_PAD // EB,),
            in_specs=[
                pl.BlockSpec((EB // 4, 128), lambda i: (i, 0)),
                pl.BlockSpec((4, EB // 4), lambda i: (0, i)),
                pl.BlockSpec((4, 128), lambda i: (0, 0)),
                pl.BlockSpec((128, 128), lambda i: (0, 0)),
                pl.BlockSpec((128, 128), lambda i: (0, 0)),
                pl.BlockSpec((1, 128), lambda i: (0, 0)),
                pl.BlockSpec((1, 128), lambda i: (0, 0)),
                pl.BlockSpec((1, 128), lambda i: (0, 0)),
            ],
            out_specs=pl.BlockSpec((EB // 2, 128), lambda i: (i, 0)),
            out_shape=jax.ShapeDtypeStruct((E_PAD // 2, 128), f32),
        )(pre4, ea4, ly["kmat"], ly["we"], ly["wo"], ly["b2"], ly["s2"],
          ly["be2"])
        ag = s2_scatter(m2.reshape(HR, HL), dst2)
        nxt = layers[li + 1] if li + 1 < len(layers) else layers[0]
        h, a, b = pl.pallas_call(
            _k3_body,
            out_shape=[jax.ShapeDtypeStruct((N, HID), f32),
                       jax.ShapeDtypeStruct((N, HL), jnp.bfloat16),
                       jax.ShapeDtypeStruct((N, HL), jnp.bfloat16)],
        )(h, ag.reshape(NC, N_PAD, HID), ly["uhT"], ly["uaT"], ly["v0u"],
          ly["u2T"], ly["b2u"], ly["s2u"], ly["be2u"],
          nxt["wdT"], nxt["wsT"], nxt["v0"])

    out = pl.pallas_call(
        _k4_body,
        out_shape=jax.ShapeDtypeStruct((G, 1), f32),
    )(h, batch.reshape(1, N), woutT, bout)
    return out.reshape(-1)


# speedup vs baseline: 9.5573x; 1.0282x over previous
import functools

import jax
import jax.numpy as jnp
import numpy as np
from jax import lax
from jax.experimental import pallas as pl
from jax.experimental.pallas import tpu as pltpu
from jax.experimental.pallas import tpu_sc as plsc

N = 10000
E = 320000
IN_DIM = 128
HID = 64
HL = 32
G = 64

NC = 2
NS = 16
NW = NC * NS

EW = 10240
E_PAD = NW * EW
SCH = 512
NSCH = EW // SCH
HR = 2 * E_PAD
HRW = HR // NW
SCH2 = 1024
NSCH2 = HRW // SCH2
N_PAD = 10240
RPT = N_PAD // NS
EB = 16384

EWH = EW // 2
NSCH_H = EWH // SCH
HRW_H = HRW // 2
NSCH2_H = HRW_H // SCH2
NBLK_H = E_PAD // EB // 2


def _s1_gather_body(a_hbm, b_hbm, dst_hbm, src_hbm, pre_hbm,
                    idxd, idxs,
                    bufa0, bufb0, bufa1, bufb1, bufo0, bufo1,
                    taba, tabb,
                    sgi, sga0, sgb0, sga1, sgb1, so0, so1, *, hofs):
    sid = lax.axis_index("s")
    wid = sid * NC + lax.axis_index("c")
    base_w = wid * EWH
    gbase_w = hofs + base_w
    tr0 = pl.multiple_of(sid * 625, 25)
    pltpu.async_copy(a_hbm.at[pl.ds(tr0, 625)], taba.at[pl.ds(tr0, 625)],
                     sgi)
    pltpu.async_copy(b_hbm.at[pl.ds(tr0, 625)], tabb.at[pl.ds(tr0, 625)],
                     sgi)
    bufa = [bufa0, bufa1]
    bufb = [bufb0, bufb1]
    bufo = [bufo0, bufo1]
    sga = [sga0, sga1]
    sgb = [sgb0, sgb1]
    so = [so0, so1]

    rb_w = pl.multiple_of(gbase_w // 128, 8)
    cp1 = pltpu.async_copy(dst_hbm.at[pl.ds(rb_w, EWH // 128)], idxd, sgi)
    cp2 = pltpu.async_copy(src_hbm.at[pl.ds(rb_w, EWH // 128)], idxs, sgi)
    pltpu.make_async_copy(a_hbm.at[pl.ds(0, 625)],
                          taba.at[pl.ds(0, 625)], sgi).wait()
    pltpu.make_async_copy(b_hbm.at[pl.ds(0, 625)],
                          tabb.at[pl.ds(0, 625)], sgi).wait()
    cp1.wait()
    cp2.wait()
    plsc.subcore_barrier()

    def fetch(i, s):
        rofs = i * (SCH // 128)
        for j in range(SCH // 128):
            pltpu.async_copy(taba.at[idxd.at[rofs + j]],
                             bufa[s].at[pl.ds(j * 128, 128)], sga[s])
            pltpu.async_copy(tabb.at[idxs.at[rofs + j]],
                             bufb[s].at[pl.ds(j * 128, 128)], sgb[s])

    def wait_fetch(s):
        pltpu.make_async_copy(a_hbm.at[pl.ds(0, SCH)], bufa[s], sga[s]).wait()
        pltpu.make_async_copy(b_hbm.at[pl.ds(0, SCH)], bufb[s], sgb[s]).wait()

    def combine(s):
        ba, bb, bo = bufa[s], bufb[s], bufo[s]

        def group(g, _):
            j0 = g * 16
            r0 = g * 4
            for i in range(16):
                j = j0 + i
                c = (i % 4) * 32
                r = r0 + i // 4
                t = ba[j, pl.ds(0, 32)] + bb[j, pl.ds(0, 32)]
                lo, hi = plsc.unpack(t, format=plsc.PackFormat.INTERLEAVED)
                bo[r, pl.ds(c, 16)] = lo
                bo[r, pl.ds(c + 16, 16)] = hi
            return 0

        lax.fori_loop(0, SCH // 16, group, 0)

    def put(i, s):
        base = pl.multiple_of(base_w + i * SCH, SCH)
        pltpu.async_copy(bufo[s], pre_hbm.at[pl.ds(base // 4, SCH // 4)],
                         so[s])

    def drain_put(s):
        pltpu.make_async_copy(pre_hbm.at[pl.ds(0, SCH // 4)],
                              bufo[s], so[s]).wait()

    fetch(0, 0)

    def pair(ii, _):
        i = 2 * ii
        fetch(i + 1, 1)
        wait_fetch(0)
        combine(0)

        @pl.when(ii > 0)
        def _():
            drain_put(0)

        put(i, 0)

        @pl.when(ii + 1 < NSCH_H // 2)
        def _():
            fetch(i + 2, 0)

        wait_fetch(1)
        combine(1)

        @pl.when(ii > 0)
        def _():
            drain_put(1)

        put(i + 1, 1)
        return 0

    lax.fori_loop(0, NSCH_H // 2, pair, 0)
    drain_put(0)
    drain_put(1)


def _s2_scatter_body(m_hbm, dst_hbm, out_hbm,
                     idx, mbuf0, mbuf1, zbuf, acc,
                     sgi, sm0, sm1, ssc, *, hofs):
    cid = lax.axis_index("c")
    sid = lax.axis_index("s")
    wid = cid * NS + sid
    base_w = wid * HRW_H
    gbase_w = hofs + base_w
    mbuf = [mbuf0, mbuf1]
    sm = [sm0, sm1]

    rb_w = pl.multiple_of(gbase_w // 128, 8)
    pltpu.async_copy(dst_hbm.at[pl.ds(rb_w, HRW_H // 128)], idx, sgi).wait()

    def zrow(j, _):
        zbuf[j, pl.ds(0, 16)] = jnp.zeros((16,), jnp.float32)
        zbuf[j, pl.ds(16, 16)] = jnp.zeros((16,), jnp.float32)
        return 0

    lax.fori_loop(0, 64, zrow, 0)

    def zcopy(k, _):
        pltpu.sync_copy(
            zbuf,
            acc.at[pl.ds(pl.multiple_of(sid * 2 * RPT + k * 64, 64), 64)])
        return 0

    lax.fori_loop(0, 2 * RPT // 64, zcopy, 0)
    plsc.subcore_barrier()

    def fetch(i, s):
        base = pl.multiple_of(base_w + i * SCH2, SCH2)
        pltpu.async_copy(m_hbm.at[pl.ds(base, SCH2)], mbuf[s], sm[s])

    def wait_fetch(s):
        pltpu.make_async_copy(m_hbm.at[pl.ds(0, SCH2)], mbuf[s],
                              sm[s]).wait()

    def scatter(i, s):
        rofs = i * (SCH2 // 128)
        for j in range(SCH2 // 128):
            pltpu.async_copy(mbuf[s].at[pl.ds(j * 128, 128)],
                             acc.at[idx.at[rofs + j]], ssc, add=True)
        pltpu.make_async_copy(m_hbm.at[pl.ds(0, SCH2)], mbuf[s],
                              ssc).wait()

    fetch(0, 0)

    def pair(ii, _):
        i = 2 * ii
        fetch(i + 1, 1)
        wait_fetch(0)
        scatter(i, 0)

        @pl.when(ii + 1 < NSCH2_H // 2)
        def _():
            fetch(i + 2, 0)

        wait_fetch(1)
        scatter(i + 1, 1)
        return 0

    lax.fori_loop(0, NSCH2_H // 2, pair, 0)
    plsc.subcore_barrier()
    srow = pl.multiple_of(sid * 2 * RPT, 2 * RPT)
    pltpu.sync_copy(acc.at[pl.ds(srow, 2 * RPT)],
                    out_hbm.at[cid, pl.ds(srow, 2 * RPT)])


@functools.cache
def _sc_kernels():
    mesh = plsc.VectorSubcoreMesh(
        core_axis_name="c", subcore_axis_name="s",
        num_cores=NC, num_subcores=NS)
    scp = pltpu.CompilerParams(use_tc_tiling_on_sc=False,
                               needs_layout_passes=False)
    i32, f32 = jnp.int32, jnp.float32
    s1 = []
    s2 = []
    for h in range(2):
        s1.append(pl.kernel(
            functools.partial(_s1_gather_body, hofs=h * (E_PAD // 2)),
            out_type=jax.ShapeDtypeStruct((E_PAD // 8, 128), f32),
            mesh=mesh,
            compiler_params=scp,
            scratch_types=(
                [pltpu.VMEM((EWH // 128, 128), i32) for _ in range(2)]
                + [pltpu.VMEM((SCH, HL), jnp.bfloat16) for _ in range(4)]
                + [pltpu.VMEM((SCH // 4, 128), f32) for _ in range(2)]
                + [pltpu.VMEM_SHARED((N, HL), jnp.bfloat16) for _ in range(2)]
                + [pltpu.SemaphoreType.DMA for _ in range(7)]
            )))
        s2.append(pl.kernel(
            functools.partial(_s2_scatter_body, hofs=h * (HR // 2)),
            out_type=jax.ShapeDtypeStruct((NC, 2 * N_PAD, HL), f32),
            mesh=mesh,
            compiler_params=scp,
            scratch_types=(
                [pltpu.VMEM((HRW_H // 128, 128), i32)]
                + [pltpu.VMEM((SCH2, HL), f32) for _ in range(2)]
                + [pltpu.VMEM((64, HL), f32)]
                + [pltpu.VMEM_SHARED((2 * N_PAD, HL), f32)]
                + [pltpu.SemaphoreType.DMA for _ in range(4)]
            )))
    return s1, s2


def _k1_body(x_ref, winT, bin_, wdT, wsT, v0, h_ref, a_ref, b_ref):
    h = jnp.dot(x_ref[...], winT[...],
                preferred_element_type=jnp.float32) + bin_[...]
    h_ref[...] = h
    a_ref[...] = (jnp.dot(h, wdT[...], preferred_element_type=jnp.float32)
                  + v0[...]).astype(jnp.bfloat16)
    b_ref[...] = jnp.dot(
        h, wsT[...], preferred_element_type=jnp.float32).astype(jnp.bfloat16)


def _k2_body(pre_ref, ea_ref, kmat, we, wo, b2_, s2_, be2_, m_ref, *, hoff):
    pid = pl.program_id(0)
    eterm = lax.dot_general(
        ea_ref[...], kmat[...], (((0,), (0,)), ((), ())),
        preferred_element_type=jnp.float32)
    z = jax.nn.relu(pre_ref[...] + eterm)
    rows = ((pid + hoff) * EB
            + 4 * lax.broadcasted_iota(jnp.int32, (EB // 4, 1), 0))
    valid = rows < E
    meven = s2_[...] * jax.nn.relu(
        jnp.dot(z, we[...], preferred_element_type=jnp.float32)
        + b2_[...]) + be2_[...]
    modd = s2_[...] * jax.nn.relu(
        jnp.dot(z, wo[...], preferred_element_type=jnp.float32)
        + b2_[...]) + be2_[...]
    m_ref[:EB // 4, :] = jnp.where(valid, meven, 0.0)
    m_ref[EB // 4:, :] = jnp.where(valid, modd, 0.0)


def _k3_body(h_ref, ag0_ref, ag1_ref, uhT, uaT, v0u, u2T, b2u, s2u, be2u,
             wdT, wsT, v0n, hn_ref, a_ref, b_ref):
    h = h_ref[...]
    aggr = (ag0_ref[0, :N, :] + ag0_ref[1, :N, :]
            + ag1_ref[0, :N, :] + ag1_ref[1, :N, :])
    t = jax.nn.relu(jnp.dot(h, uhT[...], preferred_element_type=jnp.float32)
                    + jnp.dot(aggr, uaT[...], preferred_element_type=jnp.float32)
                    + v0u[...])
    u2 = jnp.dot(t, u2T[...], preferred_element_type=jnp.float32) + b2u[...]
    hn = h + s2u[...] * jax.nn.relu(u2) + be2u[...]
    hn_ref[...] = hn
    a_ref[...] = (jnp.dot(hn, wdT[...], preferred_element_type=jnp.float32)
                  + v0n[...]).astype(jnp.bfloat16)
    b_ref[...] = jnp.dot(
        hn, wsT[...], preferred_element_type=jnp.float32).astype(jnp.bfloat16)


def _k4_body(h_ref, batch_ref, woutT, bout, out_ref):
    b = batch_ref[...]
    mask = (b == lax.broadcasted_iota(jnp.int32, (G, N), 0)).astype(jnp.float32)
    sums = jnp.dot(mask, h_ref[...], preferred_element_type=jnp.float32)
    counts = jnp.sum(mask, axis=1, keepdims=True)
    pooled = sums / jnp.maximum(counts, 1.0)
    out = jnp.dot(pooled, woutT[...], preferred_element_type=jnp.float32) + bout[...]
    out_ref[...] = jax.nn.relu(out)


def _fold_mlp(p):
    c = 1.0 / jnp.sqrt(jnp.float32(1.0 + 1e-5))
    s1 = p["g1"] * c
    w1 = p["lin1"]["w"]
    v0 = s1 * p["lin1"]["b"] + p["be1"]
    s2 = p["g2"] * c
    return s1, w1, v0, p["lin2"]["w"].T, p["lin2"]["b"], s2, p["be2"]


def _edge_perm():
    r = np.arange(EB // 4)
    evens = np.stack([4 * r, 4 * r + 1], 1).reshape(-1)
    odds = np.stack([4 * r + 2, 4 * r + 3], 1).reshape(-1)
    block_order = np.concatenate([evens, odds])
    return (np.arange(E_PAD // EB)[:, None] * EB
            + block_order[None, :]).reshape(-1)


_EDGE_PERM = _edge_perm()

_SWZ = np.stack([np.arange(16), np.arange(16) + 16], 1).reshape(-1)


def kernel(x, edge_attr, params, edge_index, batch):
    pad = E_PAD - E
    dst1 = jnp.concatenate([edge_index[1], jnp.zeros((pad,), jnp.int32)])
    src1 = jnp.concatenate([edge_index[0], jnp.zeros((pad,), jnp.int32)])
    dstp = dst1.reshape(E_PAD // 128, 128)
    srcp = src1.reshape(E_PAD // 128, 128)
    eap = jnp.concatenate([edge_attr.reshape(-1),
                           jnp.zeros((pad,), jnp.float32)])
    ea4 = eap.reshape(E_PAD // 4, 4).T
    dperm = dst1[jnp.asarray(_EDGE_PERM)]
    dst2 = (2 * dperm[:, None]
            + jnp.arange(2, dtype=jnp.int32)[None, :]).reshape(HR // 128, 128)

    layers = []
    for cp in params["convs"]:
        s1, w1, v0, w2T, b2, s2, be2 = _fold_mlp(cp["msg"])
        wdT = (w1[:, :HID] * s1[:, None]).T[:, _SWZ]
        wsT = (w1[:, HID:2 * HID] * s1[:, None]).T[:, _SWZ]
        v1 = s1 * w1[:, 2 * HID]
        kmat = jnp.zeros((4, 128), jnp.float32)
        for i in range(4):
            kmat = kmat.at[i, 32 * i:32 * (i + 1)].set(v1)
        we = jnp.zeros((128, 128), jnp.float32)
        we = we.at[0:32, 0:64].set(w2T).at[32:64, 64:128].set(w2T)
        wo = jnp.zeros((128, 128), jnp.float32)
        wo = wo.at[64:96, 0:64].set(w2T).at[96:128, 64:128].set(w2T)
        s1u, u1, v0u, u2T, b2u, s2u, be2u = _fold_mlp(cp["upd"])
        uhT = (u1[:, :HID] * s1u[:, None]).T
        uaT = (u1[:, HID:] * s1u[:, None]).T
        two = lambda v: jnp.concatenate([v, v]).reshape(1, 128)
        layers.append(dict(
            wdT=wdT, wsT=wsT, v0=v0[_SWZ].reshape(1, HL), kmat=kmat,
            we=we, wo=wo,
            b2=two(b2), s2=two(s2), be2=two(be2),
            uhT=uhT, uaT=uaT, v0u=v0u.reshape(1, HL), u2T=u2T,
            b2u=b2u.reshape(1, HID), s2u=s2u.reshape(1, HID),
            be2u=be2u.reshape(1, HID)))

    winT = params["lin_in"]["w"].T
    bin_ = params["lin_in"]["b"].reshape(1, HID)
    woutT = params["lin_out"]["w"].T
    bout = params["lin_out"]["b"].reshape(1, 1)

    f32 = jnp.float32
    h, a, b = pl.pallas_call(
        _k1_body,
        out_shape=[jax.ShapeDtypeStruct((N, HID), f32),
                   jax.ShapeDtypeStruct((N, HL), jnp.bfloat16),
                   jax.ShapeDtypeStruct((N, HL), jnp.bfloat16)],
    )(x, winT, bin_, layers[0]["wdT"], layers[0]["wsT"], layers[0]["v0"])

    s1k, s2k = _sc_kernels()

    def k2_call(pre4h, hoff, ly):
        return pl.pallas_call(
            functools.partial(_k2_body, hoff=hoff),
            grid=(NBLK_H,),
            in_specs=[
                pl.BlockSpec((EB // 4, 128), lambda i: (i, 0)),
                pl.BlockSpec((4, EB // 4), lambda i, h=hoff: (0, i + h)),
                pl.BlockSpec((4, 128), lambda i: (0, 0)),
                pl.BlockSpec((128, 128), lambda i: (0, 0)),
                pl.BlockSpec((128, 128), lambda i: (0, 0)),
                pl.BlockSpec((1, 128), lambda i: (0, 0)),
                pl.BlockSpec((1, 128), lambda i: (0, 0)),
                pl.BlockSpec((1, 128), lambda i: (0, 0)),
            ],
            out_specs=pl.BlockSpec((EB // 2, 128), lambda i: (i, 0)),
            out_shape=jax.ShapeDtypeStruct((E_PAD // 4, 128), f32),
        )(pre4h, ea4, ly["kmat"], ly["we"], ly["wo"], ly["b2"], ly["s2"],
          ly["be2"])

    for li, ly in enumerate(layers):
        pre0 = s1k[0](a, b, dstp, srcp)
        pre1 = s1k[1](a, b, dstp, srcp)
        m0 = k2_call(pre0, 0, ly)
        ag0 = s2k[0](m0.reshape(E_PAD, HL), dst2)
        m1 = k2_call(pre1, NBLK_H, ly)
        ag1 = s2k[1](m1.reshape(E_PAD, HL), dst2)
        nxt = layers[li + 1] if li + 1 < len(layers) else layers[0]
        h, a, b = pl.pallas_call(
            _k3_body,
            out_shape=[jax.ShapeDtypeStruct((N, HID), f32),
                       jax.ShapeDtypeStruct((N, HL), jnp.bfloat16),
                       jax.ShapeDtypeStruct((N, HL), jnp.bfloat16)],
        )(h, ag0.reshape(NC, N_PAD, HID), ag1.reshape(NC, N_PAD, HID),
          ly["uhT"], ly["uaT"], ly["v0u"],
          ly["u2T"], ly["b2u"], ly["s2u"], ly["be2u"],
          nxt["wdT"], nxt["wsT"], nxt["v0"])

    out = pl.pallas_call(
        _k4_body,
        out_shape=jax.ShapeDtypeStruct((G, 1), f32),
    )(h, batch.reshape(1, N), woutT, bout)
    return out.reshape(-1)
